# baseline (speedup 1.0000x reference)
import jax
import jax.numpy as jnp
from jax.experimental import pallas as pl
from jax.experimental.pallas import tpu as pltpu

EPS = 1e-06
MAX_P = 1.0
NORM_CONST = 256.0
START_GAMMA_MUL = 1.0
DECAY_GAMMA = 1.0 / 1.15

_NL = 1024
_LC = 256
_G = 8
_NS = 32
_SB = 5
_GB = 3


def _prep_kernel(x_ref, w_ref, coef_ref, gamma_ref, wtb_ref):
    xb = x_ref[0]
    ssq = jnp.sum(xb * xb, keepdims=True)
    gamma_ref[...] = jnp.minimum(jnp.sqrt(ssq) * coef_ref[...], EPS)[None]
    w = w_ref[...]
    e = jnp.exp(w - jnp.max(w))
    wt = e * (NORM_CONST / jnp.sum(e))
    bits = jax.lax.bitcast_convert_type(wt, jnp.int32)
    wtb_ref[...] = jax.lax.shift_right_logical(bits, 9)


def _lex_gt(xa, ca, xb, cb):
    return (xa > xb) | ((xa == xb) & (ca > cb))


def _xor_roll(arr, jg, ihm):
    r = arr.shape[0]
    if 2 * jg == r:
        return jnp.roll(arr, jg, axis=0)
    up = jnp.roll(arr, jg, axis=0)
    dn = jnp.roll(arr, -jg, axis=0)
    return jnp.where(ihm, up, dn)


def _in_pair(xk, ch, grow, jg, kg):
    ih = (grow & jg) != 0
    m = ih if kg >= _G else ih ^ ((grow & kg) != 0)
    pxk = _xor_roll(xk, jg, ih)
    pch = _xor_roll(ch, jg, ih)
    tp = _lex_gt(xk, ch, pxk, pch) ^ m
    return jnp.where(tp, pxk, xk), jnp.where(tp, pch, ch)


def _in_word(wd, grow, jg, kg):
    ih = (grow & jg) != 0
    m = ih if kg >= _G else ih ^ ((grow & kg) != 0)
    pw = _xor_roll(wd, jg, ih)
    tp = (wd > pw) ^ m
    return jnp.where(tp, pw, wd)


def _cross_pair(xs, cs, a, b, notasc, m):
    sgp = _lex_gt(xs[a], cs[a], xs[b], cs[b])
    if m is not None:
        tp = sgp ^ m
        xs[a], xs[b] = (jnp.where(tp, xs[b], xs[a]),
                        jnp.where(tp, xs[a], xs[b]))
        cs[a], cs[b] = (jnp.where(tp, cs[b], cs[a]),
                        jnp.where(tp, cs[a], cs[b]))
    elif not notasc:
        xs[a], xs[b] = (jnp.where(sgp, xs[b], xs[a]),
                        jnp.where(sgp, xs[a], xs[b]))
        cs[a], cs[b] = (jnp.where(sgp, cs[b], cs[a]),
                        jnp.where(sgp, cs[a], cs[b]))
    else:
        xs[a], xs[b] = (jnp.where(sgp, xs[a], xs[b]),
                        jnp.where(sgp, xs[b], xs[a]))
        cs[a], cs[b] = (jnp.where(sgp, cs[a], cs[b]),
                        jnp.where(sgp, cs[b], cs[a]))


def _cross_word(ws, a, b, notasc, m):
    sgp = ws[a] > ws[b]
    if m is not None:
        tp = sgp ^ m
        ws[a], ws[b] = (jnp.where(tp, ws[b], ws[a]),
                        jnp.where(tp, ws[a], ws[b]))
    elif not notasc:
        ws[a], ws[b] = (jnp.where(sgp, ws[b], ws[a]),
                        jnp.where(sgp, ws[a], ws[b]))
    else:
        ws[a], ws[b] = (jnp.where(sgp, ws[a], ws[b]),
                        jnp.where(sgp, ws[b], ws[a]))


def _network(stage_cross, stage_in):
    k = 2
    while k <= 256:
        j = k // 2
        while j >= 1:
            if j >= _NS:
                stage_in(j >> _SB, max(k >> _SB, 1))
            else:
                for s in range(_NS):
                    if s & j == 0:
                        if k < _NS:
                            stage_cross(s, s ^ j, (s & k) != 0, None)
                        else:
                            stage_cross(s, s ^ j, False, k >> _SB)
            j //= 2
        k *= 2


def _main_kernel(x_ref, wtb_ref, gamma_ref, p_ref, out_ref):
    nl = x_ref.shape[2]
    lc = min(_LC, nl)
    grow = jax.lax.broadcasted_iota(jnp.int32, (_G, 1), 0)
    gamma = gamma_ref[0]
    expo = (jax.nn.sigmoid(p_ref[...]) * MAX_P - 2.0) * 0.5
    wtb = [wtb_ref[s * _G:(s + 1) * _G] for s in range(_NS)]
    dir_masks = {}

    def dmask(kg):
        if kg not in dir_masks:
            dir_masks[kg] = (grow & kg) != 0
        return dir_masks[kg]

    for l0 in range(0, nl, lc):
        xs, cs = [], []
        for s in range(_NS):
            xs.append(-x_ref[0, s * _G:(s + 1) * _G, l0:l0 + lc])
            cs.append((jax.lax.broadcasted_iota(jnp.int32, (_G, lc), 0)
                       + s * _G).astype(jnp.float32))

        def s1_cross(a, b, notasc, kg):
            _cross_pair(xs, cs, a, b, notasc, None if kg is None else dmask(kg))

        def s1_in(jg, kg):
            for s in range(_NS):
                xs[s], cs[s] = _in_pair(xs[s], cs[s], grow, jg, kg)

        _network(s1_cross, s1_in)

        ws = []
        for s in range(_NS):
            chi = cs[s].astype(jnp.int32)
            kap = jax.lax.shift_left(chi & (_G - 1), _SB) | jax.lax.shift_right_logical(chi, _GB)
            ws.append(jax.lax.shift_left(kap, 23) | wtb[s])

        def s2_cross(a, b, notasc, kg):
            _cross_word(ws, a, b, notasc, None if kg is None else dmask(kg))

        def s2_in(jg, kg):
            for s in range(_NS):
                ws[s] = _in_word(ws[s], grow, jg, kg)

        _network(s2_cross, s2_in)

        for s in range(_NS):
            wt_g = jax.lax.bitcast_convert_type(
                jax.lax.shift_left(ws[s] & 0x7FFFFF, 9), jnp.float32)
            xcb = x_ref[0, s * _G:(s + 1) * _G, l0:l0 + lc]
            out_ref[0, s * _G:(s + 1) * _G, l0:l0 + lc] = (
                wt_g * jnp.exp(expo * jnp.log(xcb * xcb + gamma)))


def kernel(x, weights, p, step_num):
    b, c, h, w = x.shape
    s = h * w
    xr = x.reshape(b, c, s)
    nl = min(_NL, s)

    coef = (START_GAMMA_MUL
            * jnp.power(jnp.float32(DECAY_GAMMA),
                        jnp.asarray(step_num, jnp.float32))).reshape(1, 1)
    w_col = weights.reshape(c, 1)
    p_arr = p.reshape(1, 1).astype(jnp.float32)

    gamma, wtb = pl.pallas_call(
        _prep_kernel,
        grid=(b,),
        in_specs=[
            pl.BlockSpec((1, c, s), lambda i: (i, 0, 0)),
            pl.BlockSpec((c, 1), lambda i: (0, 0)),
            pl.BlockSpec((1, 1), lambda i: (0, 0)),
        ],
        out_specs=[
            pl.BlockSpec((1, 1, 1), lambda i: (i, 0, 0)),
            pl.BlockSpec((c, 1), lambda i: (0, 0)),
        ],
        out_shape=[
            jax.ShapeDtypeStruct((b, 1, 1), jnp.float32),
            jax.ShapeDtypeStruct((c, 1), jnp.int32),
        ],
    )(xr, w_col, coef)

    wtb_perm = wtb.reshape(_G, _NS, 1).transpose(1, 0, 2).reshape(c, 1)

    out = pl.pallas_call(
        _main_kernel,
        grid=(b, s // nl),
        in_specs=[
            pl.BlockSpec((1, c, nl), lambda i, t: (i, 0, t)),
            pl.BlockSpec((c, 1), lambda i, t: (0, 0)),
            pl.BlockSpec((1, 1, 1), lambda i, t: (i, 0, 0)),
            pl.BlockSpec((1, 1), lambda i, t: (0, 0)),
        ],
        out_specs=pl.BlockSpec((1, c, nl), lambda i, t: (i, 0, t)),
        out_shape=jax.ShapeDtypeStruct((b, c, s), jnp.float32),
    )(xr, wtb_perm, gamma, p_arr)

    return out.reshape(b, c, h, w)


# speedup vs baseline: 1.1244x; 1.1244x over previous
import jax
import jax.numpy as jnp
from jax.experimental import pallas as pl
from jax.experimental.pallas import tpu as pltpu

EPS = 1e-06
MAX_P = 1.0
NORM_CONST = 256.0
START_GAMMA_MUL = 1.0
DECAY_GAMMA = 1.0 / 1.15

_NL = 1024
_LC = 128
_G = 8
_NS = 32
_SB = 5
_GB = 3


def _prep_kernel(x_ref, w_ref, coef_ref, gamma_ref, wtb_ref):
    xb = x_ref[0]
    ssq = jnp.sum(xb * xb, keepdims=True)
    gamma_ref[...] = jnp.minimum(jnp.sqrt(ssq) * coef_ref[...], EPS)[None]
    w = w_ref[...]
    e = jnp.exp(w - jnp.max(w))
    wt = e * (NORM_CONST / jnp.sum(e))
    bits = jax.lax.bitcast_convert_type(wt, jnp.int32)
    wtb_ref[...] = jax.lax.shift_right_logical(bits, 9)


def _lex_gt(xa, ca, xb, cb):
    return (xa > xb) | ((xa == xb) & (ca > cb))


def _xor_roll(arr, jg, ihm):
    r = arr.shape[0]
    if 2 * jg == r:
        return jnp.roll(arr, jg, axis=0)
    up = jnp.roll(arr, jg, axis=0)
    dn = jnp.roll(arr, -jg, axis=0)
    return jnp.where(ihm, up, dn)


def _in_pair(xk, ch, grow, jg, kg):
    ih = (grow & jg) != 0
    m = ih if kg >= _G else ih ^ ((grow & kg) != 0)
    pxk = _xor_roll(xk, jg, ih)
    pch = _xor_roll(ch, jg, ih)
    tp = _lex_gt(xk, ch, pxk, pch) ^ m
    return jnp.where(tp, pxk, xk), jnp.where(tp, pch, ch)


def _in_word(wd, grow, jg, kg):
    ih = (grow & jg) != 0
    m = ih if kg >= _G else ih ^ ((grow & kg) != 0)
    pw = _xor_roll(wd, jg, ih)
    tp = (wd > pw) ^ m
    return jnp.where(tp, pw, wd)


def _cross_pair(xs, cs, a, b, notasc, m):
    sgp = _lex_gt(xs[a], cs[a], xs[b], cs[b])
    if m is not None:
        tp = sgp ^ m
        xs[a], xs[b] = (jnp.where(tp, xs[b], xs[a]),
                        jnp.where(tp, xs[a], xs[b]))
        cs[a], cs[b] = (jnp.where(tp, cs[b], cs[a]),
                        jnp.where(tp, cs[a], cs[b]))
    elif not notasc:
        xs[a], xs[b] = (jnp.minimum(xs[a], xs[b]),
                        jnp.maximum(xs[a], xs[b]))
        cs[a], cs[b] = (jnp.where(sgp, cs[b], cs[a]),
                        jnp.where(sgp, cs[a], cs[b]))
    else:
        xs[a], xs[b] = (jnp.maximum(xs[a], xs[b]),
                        jnp.minimum(xs[a], xs[b]))
        cs[a], cs[b] = (jnp.where(sgp, cs[a], cs[b]),
                        jnp.where(sgp, cs[b], cs[a]))


def _cross_word(ws, a, b, notasc, m):
    sgp = ws[a] > ws[b]
    if m is not None:
        tp = sgp ^ m
        ws[a], ws[b] = (jnp.where(tp, ws[b], ws[a]),
                        jnp.where(tp, ws[a], ws[b]))
    elif not notasc:
        ws[a], ws[b] = jnp.minimum(ws[a], ws[b]), jnp.maximum(ws[a], ws[b])
    else:
        ws[a], ws[b] = jnp.maximum(ws[a], ws[b]), jnp.minimum(ws[a], ws[b])


def _network(stage_cross, stage_in):
    k = 2
    while k <= 256:
        j = k // 2
        while j >= 1:
            if j >= _NS:
                stage_in(j >> _SB, max(k >> _SB, 1))
            else:
                for s in range(_NS):
                    if s & j == 0:
                        if k < _NS:
                            stage_cross(s, s ^ j, (s & k) != 0, None)
                        else:
                            stage_cross(s, s ^ j, False, k >> _SB)
            j //= 2
        k *= 2


def _main_kernel(x_ref, wtb_ref, gamma_ref, p_ref, out_ref):
    nl = x_ref.shape[2]
    lc = min(_LC, nl)
    grow = jax.lax.broadcasted_iota(jnp.int32, (_G, 1), 0)
    gamma = gamma_ref[0]
    expo = (jax.nn.sigmoid(p_ref[...]) * MAX_P - 2.0) * 0.5
    wtb = [wtb_ref[s * _G:(s + 1) * _G] for s in range(_NS)]
    dir_masks = {}

    def dmask(kg):
        if kg not in dir_masks:
            dir_masks[kg] = (grow & kg) != 0
        return dir_masks[kg]

    for l0 in range(0, nl, lc):
        xs, cs = [], []
        for s in range(_NS):
            xs.append(-x_ref[0, s * _G:(s + 1) * _G, l0:l0 + lc])
            cs.append((jax.lax.broadcasted_iota(jnp.int32, (_G, lc), 0)
                       + s * _G).astype(jnp.float32))

        def s1_cross(a, b, notasc, kg):
            _cross_pair(xs, cs, a, b, notasc, None if kg is None else dmask(kg))

        def s1_in(jg, kg):
            for s in range(_NS):
                xs[s], cs[s] = _in_pair(xs[s], cs[s], grow, jg, kg)

        _network(s1_cross, s1_in)

        ws = []
        for s in range(_NS):
            chi = cs[s].astype(jnp.int32)
            kap = jax.lax.shift_left(chi & (_G - 1), _SB) | jax.lax.shift_right_logical(chi, _GB)
            ws.append(jax.lax.shift_left(kap, 23) | wtb[s])

        def s2_cross(a, b, notasc, kg):
            _cross_word(ws, a, b, notasc, None if kg is None else dmask(kg))

        def s2_in(jg, kg):
            for s in range(_NS):
                ws[s] = _in_word(ws[s], grow, jg, kg)

        _network(s2_cross, s2_in)

        for s in range(_NS):
            wt_g = jax.lax.bitcast_convert_type(
                jax.lax.shift_left(ws[s] & 0x7FFFFF, 9), jnp.float32)
            xcb = x_ref[0, s * _G:(s + 1) * _G, l0:l0 + lc]
            out_ref[0, s * _G:(s + 1) * _G, l0:l0 + lc] = (
                wt_g * jnp.exp(expo * jnp.log(xcb * xcb + gamma)))


def kernel(x, weights, p, step_num):
    b, c, h, w = x.shape
    s = h * w
    xr = x.reshape(b, c, s)
    nl = min(_NL, s)

    coef = (START_GAMMA_MUL
            * jnp.power(jnp.float32(DECAY_GAMMA),
                        jnp.asarray(step_num, jnp.float32))).reshape(1, 1)
    w_col = weights.reshape(c, 1)
    p_arr = p.reshape(1, 1).astype(jnp.float32)

    gamma, wtb = pl.pallas_call(
        _prep_kernel,
        grid=(b,),
        in_specs=[
            pl.BlockSpec((1, c, s), lambda i: (i, 0, 0)),
            pl.BlockSpec((c, 1), lambda i: (0, 0)),
            pl.BlockSpec((1, 1), lambda i: (0, 0)),
        ],
        out_specs=[
            pl.BlockSpec((1, 1, 1), lambda i: (i, 0, 0)),
            pl.BlockSpec((c, 1), lambda i: (0, 0)),
        ],
        out_shape=[
            jax.ShapeDtypeStruct((b, 1, 1), jnp.float32),
            jax.ShapeDtypeStruct((c, 1), jnp.int32),
        ],
    )(xr, w_col, coef)

    wtb_perm = wtb.reshape(_G, _NS, 1).transpose(1, 0, 2).reshape(c, 1)

    out = pl.pallas_call(
        _main_kernel,
        grid=(b, s // nl),
        in_specs=[
            pl.BlockSpec((1, c, nl), lambda i, t: (i, 0, t)),
            pl.BlockSpec((c, 1), lambda i, t: (0, 0)),
            pl.BlockSpec((1, 1, 1), lambda i, t: (i, 0, 0)),
            pl.BlockSpec((1, 1), lambda i, t: (0, 0)),
        ],
        out_specs=pl.BlockSpec((1, c, nl), lambda i, t: (i, 0, t)),
        out_shape=jax.ShapeDtypeStruct((b, c, s), jnp.float32),
    )(xr, wtb_perm, gamma, p_arr)

    return out.reshape(b, c, h, w)
